# baseline (speedup 1.0000x reference)
import numpy as np

import jax
import jax.numpy as jnp
from jax.experimental import pallas as pl
from jax.experimental.pallas import tpu as pltpu

B = 128
V = 100000
VC = 20480
NBLK = (V + VC - 1) // VC

_NEG_INF = float("-inf")


def _gumbel_constant() -> np.ndarray:
    def rotl(x, d):
        return (x << np.uint32(d)) | (x >> np.uint32(32 - d))

    rot = [np.uint32([13, 15, 26, 6]), np.uint32([17, 29, 16, 24])]
    k1, k2 = np.uint32(0), np.uint32(42)
    ks = [k1, k2, np.uint32(k1 ^ k2 ^ np.uint32(0x1BD11BDA))]
    x = [np.uint32(0) + ks[0], np.arange(B * V, dtype=np.uint32) + ks[1]]
    with np.errstate(over="ignore"):
        for i in range(5):
            for r in rot[i % 2]:
                x[0] = x[0] + x[1]
                x[1] = rotl(x[1], int(r))
                x[1] = x[0] ^ x[1]
            x[0] = x[0] + ks[(i + 1) % 3]
            x[1] = x[1] + ks[(i + 2) % 3] + np.uint32(i + 1)
    bits = x[0] ^ x[1]
    float_bits = (bits >> np.uint32(9)) | np.uint32(0x3F800000)
    floats = float_bits.view(np.float32) - np.float32(1.0)
    tiny = np.float32(np.finfo(np.float32).tiny)
    u = np.maximum(tiny, floats * (np.float32(1.0) - tiny) + tiny)
    g = (-np.log(-np.log(u))).astype(np.float32).reshape(B, V)
    g[:, 0] = _NEG_INF
    return g


_GUMBEL = _gumbel_constant()


def _sample_kernel(x_ref, g_ref, out_ref, m_scr, a_scr):
    i = pl.program_id(0)

    @pl.when(i == 0)
    def _init():
        m_scr[...] = jnp.full((B, 1), _NEG_INF, jnp.float32)
        a_scr[...] = jnp.zeros((B, 1), jnp.int32)

    s = x_ref[0] + g_ref[...]
    col = jax.lax.broadcasted_iota(jnp.int32, (B, VC), 1) + i * VC
    s = jnp.where(col < V, s, _NEG_INF)
    loc_max = jnp.max(s, axis=1, keepdims=True)
    cand = jnp.where(s == loc_max, col, V)
    loc_arg = jnp.min(cand, axis=1, keepdims=True)
    better = loc_max > m_scr[...]
    a_scr[...] = jnp.where(better, loc_arg, a_scr[...])
    m_scr[...] = jnp.maximum(loc_max, m_scr[...])

    @pl.when(i == NBLK - 1)
    def _fin():
        out_ref[...] = a_scr[...]


def kernel(decoder_out):
    return pl.pallas_call(
        _sample_kernel,
        grid=(NBLK,),
        in_specs=[
            pl.BlockSpec((1, B, VC), lambda i: (0, 0, i)),
            pl.BlockSpec((B, VC), lambda i: (0, i)),
        ],
        out_specs=pl.BlockSpec((B, 1), lambda i: (0, 0)),
        out_shape=jax.ShapeDtypeStruct((B, 1), jnp.int32),
        scratch_shapes=[
            pltpu.VMEM((B, 1), jnp.float32),
            pltpu.VMEM((B, 1), jnp.int32),
        ],
    )(decoder_out, _GUMBEL)


# speedup vs baseline: 1.0193x; 1.0193x over previous
import numpy as np

import jax
import jax.numpy as jnp
from jax.experimental import pallas as pl
from jax.experimental.pallas import tpu as pltpu

B = 128
V = 100000
VC = 10240
NBLK = (V + VC - 1) // VC

_NEG_INF = float("-inf")


def _gumbel_constant() -> np.ndarray:
    def rotl(x, d):
        return (x << np.uint32(d)) | (x >> np.uint32(32 - d))

    rot = [np.uint32([13, 15, 26, 6]), np.uint32([17, 29, 16, 24])]
    k1, k2 = np.uint32(0), np.uint32(42)
    ks = [k1, k2, np.uint32(k1 ^ k2 ^ np.uint32(0x1BD11BDA))]
    x = [np.uint32(0) + ks[0], np.arange(B * V, dtype=np.uint32) + ks[1]]
    with np.errstate(over="ignore"):
        for i in range(5):
            for r in rot[i % 2]:
                x[0] = x[0] + x[1]
                x[1] = rotl(x[1], int(r))
                x[1] = x[0] ^ x[1]
            x[0] = x[0] + ks[(i + 1) % 3]
            x[1] = x[1] + ks[(i + 2) % 3] + np.uint32(i + 1)
    bits = x[0] ^ x[1]
    float_bits = (bits >> np.uint32(9)) | np.uint32(0x3F800000)
    floats = float_bits.view(np.float32) - np.float32(1.0)
    tiny = np.float32(np.finfo(np.float32).tiny)
    u = np.maximum(tiny, floats * (np.float32(1.0) - tiny) + tiny)
    g = (-np.log(-np.log(u))).astype(np.float32).reshape(B, V)
    g[:, 0] = _NEG_INF
    return g


_GUMBEL = _gumbel_constant()


def _sample_kernel(x_ref, g_ref, out_ref, m_scr, a_scr):
    i = pl.program_id(0)

    @pl.when(i == 0)
    def _init():
        m_scr[...] = jnp.full((B, 1), _NEG_INF, jnp.float32)
        a_scr[...] = jnp.zeros((B, 1), jnp.int32)

    s = x_ref[0] + g_ref[...]
    col = jax.lax.broadcasted_iota(jnp.int32, (B, VC), 1) + i * VC
    s = jnp.where(col < V, s, _NEG_INF)
    loc_max = jnp.max(s, axis=1, keepdims=True)
    cand = jnp.where(s == loc_max, col, V)
    loc_arg = jnp.min(cand, axis=1, keepdims=True)
    better = loc_max > m_scr[...]
    a_scr[...] = jnp.where(better, loc_arg, a_scr[...])
    m_scr[...] = jnp.maximum(loc_max, m_scr[...])

    @pl.when(i == NBLK - 1)
    def _fin():
        out_ref[...] = a_scr[...]


def kernel(decoder_out):
    return pl.pallas_call(
        _sample_kernel,
        grid=(NBLK,),
        in_specs=[
            pl.BlockSpec((1, B, VC), lambda i: (0, 0, i)),
            pl.BlockSpec((B, VC), lambda i: (0, i)),
        ],
        out_specs=pl.BlockSpec((B, 1), lambda i: (0, 0)),
        out_shape=jax.ShapeDtypeStruct((B, 1), jnp.int32),
        scratch_shapes=[
            pltpu.VMEM((B, 1), jnp.float32),
            pltpu.VMEM((B, 1), jnp.int32),
        ],
    )(decoder_out, _GUMBEL)
